# Initial kernel scaffold; baseline (speedup 1.0000x reference)
import jax
import jax.numpy as jnp
from jax.experimental import pallas as pl


def kernel(x, edge_attr, pos, params, edge_index, batch):
    raise NotImplementedError("write your pallas kernel here")


# speedup vs baseline: 4.5766x; 4.5766x over previous
import functools

import jax
import jax.numpy as jnp
import numpy as np
from jax import lax
from jax.experimental import pallas as pl
from jax.experimental.pallas import tpu as pltpu
from jax.experimental.pallas import tpu_sc as plsc

N = 10000
NF_DIM = 128
NF_EMB = 195
EDGE_ATTR_DIM = 4
E0 = 320000
E = E0 + N
E_PAD = 331776
CH = 128
NW = 32
G = E_PAD // (NW * CH)
EBLK = 1024
RBLK = 2000
NC, NS = 2, 16
N_PAD = 10112
ROWS_PER_SUB = N_PAD // NS

EMB_IDXS = [1, 4, 5, 7, 8]
EMB_NUMS = [40, 8, 2, 2, 9]
NCNT = float(N * NF_EMB)


def _silu(x):
    return x * jax.nn.sigmoid(x)


def _init_body(x_ref, e0, e1, e2, e3, e4, feats_ref, colsum_ref):
    i = pl.program_id(0)
    x = x_ref[...]
    embs = [e0[...], e1[...], e2[...], e3[...], e4[...]]
    pieces = [x[:, 0:1], x[:, 2:4], x[:, 6:7], x[:, 9:128]]
    for k, (col, n) in enumerate(zip(EMB_IDXS, EMB_NUMS)):
        idx = x[:, col:col + 1].astype(jnp.int32)
        io = lax.broadcasted_iota(jnp.int32, (RBLK, n), 1)
        oh = (io == idx).astype(jnp.float32)
        pieces.append(jnp.dot(oh, embs[k], preferred_element_type=jnp.float32))
    feats = jnp.concatenate(pieces, axis=1)
    feats_ref[...] = feats
    cs = jnp.sum(feats, axis=0, keepdims=True)

    @pl.when(i == 0)
    def _():
        colsum_ref[...] = cs

    @pl.when(i > 0)
    def _():
        colsum_ref[...] += cs


def _init_call(x, embs):
    grid = N // RBLK
    bs_w = lambda s: pl.BlockSpec(s, lambda i: (0, 0))
    return pl.pallas_call(
        _init_body,
        grid=(grid,),
        in_specs=[pl.BlockSpec((RBLK, NF_DIM), lambda i: (i, 0))]
        + [bs_w(e.shape) for e in embs],
        out_specs=[pl.BlockSpec((RBLK, NF_EMB), lambda i: (i, 0)),
                   pl.BlockSpec((1, NF_EMB), lambda i: (0, 0))],
        out_shape=[jax.ShapeDtypeStruct((N, NF_EMB), jnp.float32),
                   jax.ShapeDtypeStruct((1, NF_EMB), jnp.float32)],
    )(x, *embs)


def _var_body(f_ref, cs_ref, s2_ref):
    i = pl.program_id(0)
    mu = jnp.sum(cs_ref[...]) * (1.0 / NCNT)
    xc = f_ref[...] - mu
    p = jnp.sum(xc * xc).reshape(1, 1)

    @pl.when(i == 0)
    def _():
        s2_ref[...] = p

    @pl.when(i > 0)
    def _():
        s2_ref[...] += p


def _var_call(feats, colsum):
    return pl.pallas_call(
        _var_body,
        grid=(N // RBLK,),
        in_specs=[pl.BlockSpec((RBLK, NF_EMB), lambda i: (i, 0)),
                  pl.BlockSpec((1, NF_EMB), lambda i: (0, 0))],
        out_specs=pl.BlockSpec((1, 1), lambda i: (0, 0)),
        out_shape=jax.ShapeDtypeStruct((1, 1), jnp.float32),
    )(feats, colsum)


def _tables_body(f_ref, c_ref, cs_ref, s2_ref, lnw, lnb, wi, wj, nw1t, nb1,
                 tdst_ref, tsrc_ref, hfw_ref):
    mu = jnp.sum(cs_ref[...]) * (1.0 / NCNT)
    var = s2_ref[0, 0] * (1.0 / NCNT)
    rstd = jax.lax.rsqrt(var + 1e-5)
    f = f_ref[...]
    hf = (f - mu) * rstd * lnw[...] + lnb[...]
    c128 = jnp.concatenate(
        [c_ref[...], jnp.zeros((RBLK, 112), jnp.float32)], axis=1)
    cu = lax.bitcast_convert_type(
        c128.astype(jnp.bfloat16), jnp.uint16).astype(jnp.uint32)
    pi = jnp.dot(f, wi[...], preferred_element_type=jnp.float32)
    pj = jnp.dot(f, wj[...], preferred_element_type=jnp.float32)
    piu = lax.bitcast_convert_type(
        pi.astype(jnp.bfloat16), jnp.uint16).astype(jnp.uint32)
    pju = lax.bitcast_convert_type(
        pj.astype(jnp.bfloat16), jnp.uint16).astype(jnp.uint32)
    tdst_ref[...] = lax.bitcast_convert_type(piu | (cu << 16), jnp.int32)
    tsrc_ref[...] = lax.bitcast_convert_type(pju | (cu << 16), jnp.int32)
    hfw_ref[...] = jnp.dot(hf, nw1t[...], preferred_element_type=jnp.float32) + nb1[...]


def _tables_call(feats, coors, colsum, s2, lnw, lnb, wi, wj, nw1t, nb1):
    bs_w = lambda a: pl.BlockSpec(a.shape, lambda i: tuple(0 for _ in a.shape))
    return pl.pallas_call(
        _tables_body,
        grid=(N // RBLK,),
        in_specs=[pl.BlockSpec((RBLK, NF_EMB), lambda i: (i, 0)),
                  pl.BlockSpec((RBLK, 16), lambda i: (i, 0)),
                  bs_w(colsum), bs_w(s2), bs_w(lnw), bs_w(lnb),
                  bs_w(wi), bs_w(wj), bs_w(nw1t), bs_w(nb1)],
        out_specs=[pl.BlockSpec((RBLK, 128), lambda i: (i, 0)),
                   pl.BlockSpec((RBLK, 128), lambda i: (i, 0)),
                   pl.BlockSpec((RBLK, 128), lambda i: (i, 0))],
        out_shape=[jax.ShapeDtypeStruct((N, 128), jnp.int32),
                   jax.ShapeDtypeStruct((N, 128), jnp.int32),
                   jax.ShapeDtypeStruct((N, 128), jnp.float32)],
    )(feats, coors, colsum, s2, lnw, lnb, wi, wj, nw1t, nb1)


def _gather_body(tdst, tsrc, dsti, srci, gd, gs,
                 idxd_v, idxs_v, bufd, bufs, semd, sems):
    c = lax.axis_index("c")
    s = lax.axis_index("s")
    wid = s * NC + c
    pltpu.sync_copy(dsti.at[wid], idxd_v)
    pltpu.sync_copy(srci.at[wid], idxs_v)
    base = wid * G * CH

    def step(g, carry):
        cpd = pltpu.async_copy(tdst.at[idxd_v.at[g]], bufd, semd)
        cps = pltpu.async_copy(tsrc.at[idxs_v.at[g]], bufs, sems)
        cpd.wait()
        cps.wait()
        pltpu.sync_copy(bufd, gd.at[pl.ds(base + g * CH, CH)])
        pltpu.sync_copy(bufs, gs.at[pl.ds(base + g * CH, CH)])
        return carry

    lax.fori_loop(0, G, step, 0)


@functools.partial(
    pl.kernel,
    out_type=[jax.ShapeDtypeStruct((E_PAD, 128), jnp.int32),
              jax.ShapeDtypeStruct((E_PAD, 128), jnp.int32)],
    mesh=plsc.VectorSubcoreMesh(core_axis_name="c", subcore_axis_name="s"),
    scratch_types=[pltpu.VMEM((G, CH), jnp.int32),
                   pltpu.VMEM((G, CH), jnp.int32),
                   pltpu.VMEM((CH, 128), jnp.int32),
                   pltpu.VMEM((CH, 128), jnp.int32),
                   pltpu.SemaphoreType.DMA,
                   pltpu.SemaphoreType.DMA],
)
def _gather_call(tdst, tsrc, dsti, srci, gd, gs, *scratch):
    _gather_body(tdst, tsrc, dsti, srci, gd, gs, *scratch)


def _edge_body(gd_ref, gs_ref, ea_ref, wa, b1, wdrow, ew2, eb2,
               cw1, cb1, cw2, cb2, out_ref):
    i = pl.program_id(0)
    gdw = lax.bitcast_convert_type(gd_ref[...], jnp.uint32)
    gsw = lax.bitcast_convert_type(gs_ref[...], jnp.uint32)

    def _lo(w):
        return lax.bitcast_convert_type(
            (w & 0xFFFF).astype(jnp.uint16), jnp.bfloat16).astype(jnp.float32)

    def _hi(w):
        return lax.bitcast_convert_type(
            (w >> 16).astype(jnp.uint16), jnp.bfloat16).astype(jnp.float32)

    pd = _lo(gdw)
    ps = _lo(gsw)
    rel = _hi(gsw) - _hi(gdw)
    rd = jnp.sum(rel * rel, axis=1, keepdims=True)
    pre = (pd + ps
           + jnp.dot(ea_ref[...], wa[...], preferred_element_type=jnp.float32)
           + rd * wdrow[...] + b1[...])
    h = _silu(pre)
    m = _silu(jnp.dot(h, ew2[...], preferred_element_type=jnp.float32) + eb2[...])
    t = _silu(jnp.dot(m, cw1[...], preferred_element_type=jnp.float32) + cb1[...])
    cw16 = jnp.dot(t, cw2[...], preferred_element_type=jnp.float32) + cb2[...]
    cw = cw16[:, 0:1]
    eid = i * EBLK + lax.broadcasted_iota(jnp.int32, (EBLK, 1), 0)
    valid = (eid < E).astype(jnp.float32)
    out = jnp.concatenate(
        [m, cw * rel[:, 0:3], jnp.zeros((EBLK, 93), jnp.float32)], axis=1)
    out_ref[...] = out * valid


def _edge_call(gd, gs, ea, wa, b1, wdrow, ew2, eb2, cw1, cb1, cw2, cb2):
    ws = [wa, b1, wdrow, ew2, eb2, cw1, cb1, cw2, cb2]
    bs_w = lambda a: pl.BlockSpec(a.shape, lambda i: tuple(0 for _ in a.shape))
    return pl.pallas_call(
        _edge_body,
        grid=(E_PAD // EBLK,),
        in_specs=[pl.BlockSpec((EBLK, 128), lambda i: (i, 0)),
                  pl.BlockSpec((EBLK, 128), lambda i: (i, 0)),
                  pl.BlockSpec((EBLK, EDGE_ATTR_DIM), lambda i: (i, 0))]
        + [bs_w(a) for a in ws],
        out_specs=pl.BlockSpec((EBLK, 128), lambda i: (i, 0)),
        out_shape=jax.ShapeDtypeStruct((E_PAD, 128), jnp.float32),
    )(gd, gs, ea, *ws)


def _scatter_body(m_hbm, dsti, zeros_hbm, out_hbm, idx_v, mbuf, acc_sh):
    c = lax.axis_index("c")
    s = lax.axis_index("s")
    r0 = s * ROWS_PER_SUB
    pltpu.sync_copy(zeros_hbm.at[pl.ds(r0, ROWS_PER_SUB)],
                    acc_sh.at[pl.ds(r0, ROWS_PER_SUB)])
    plsc.subcore_barrier()
    wid = c * NS + s
    pltpu.sync_copy(dsti.at[wid], idx_v)
    base = wid * G * CH

    def step(g, carry):
        pltpu.sync_copy(m_hbm.at[pl.ds(base + g * CH, CH)], mbuf)
        pltpu.sync_copy(mbuf, acc_sh.at[idx_v.at[g]], add=True)
        return carry

    lax.fori_loop(0, G, step, 0)
    plsc.subcore_barrier()
    pltpu.sync_copy(acc_sh.at[pl.ds(r0, ROWS_PER_SUB)],
                    out_hbm.at[c, pl.ds(r0, ROWS_PER_SUB)])


@functools.partial(
    pl.kernel,
    out_type=jax.ShapeDtypeStruct((2, N_PAD, 128), jnp.float32),
    mesh=plsc.VectorSubcoreMesh(core_axis_name="c", subcore_axis_name="s"),
    scratch_types=[pltpu.VMEM((G, CH), jnp.int32),
                   pltpu.VMEM((CH, 128), jnp.float32),
                   pltpu.VMEM_SHARED((N_PAD, 128), jnp.float32)],
)
def _scatter_call(m_hbm, dsti, zeros_hbm, out_hbm, *scratch):
    _scatter_body(m_hbm, dsti, zeros_hbm, out_hbm, *scratch)


def _update_body(f_ref, a0_ref, a1_ref, c_ref, hfw_ref, nw1b, nw2, nb2,
                 fo_ref, co_ref, colsum_ref):
    i = pl.program_id(0)
    acc = a0_ref[...] + a1_ref[...]
    m_i = acc[:, :32]
    mh16 = acc[:, 32:48]
    h1 = _silu(hfw_ref[...]
               + jnp.dot(m_i, nw1b[...], preferred_element_type=jnp.float32))
    h2 = jnp.dot(h1, nw2[...], preferred_element_type=jnp.float32) + nb2[...]
    fo = f_ref[...] + h2
    fo_ref[...] = fo
    co_ref[...] = c_ref[...] + mh16 * (1.0 / N)
    cs = jnp.sum(fo, axis=0, keepdims=True)

    @pl.when(i == 0)
    def _():
        colsum_ref[...] = cs

    @pl.when(i > 0)
    def _():
        colsum_ref[...] += cs


def _update_call(feats, acc0, acc1, coors, hfw, nw1b, nw2, nb2):
    ws = [nw1b, nw2, nb2]
    bs_w = lambda a: pl.BlockSpec(a.shape, lambda i: tuple(0 for _ in a.shape))
    return pl.pallas_call(
        _update_body,
        grid=(N // RBLK,),
        in_specs=[pl.BlockSpec((RBLK, NF_EMB), lambda i: (i, 0)),
                  pl.BlockSpec((RBLK, 128), lambda i: (i, 0)),
                  pl.BlockSpec((RBLK, 128), lambda i: (i, 0)),
                  pl.BlockSpec((RBLK, 16), lambda i: (i, 0)),
                  pl.BlockSpec((RBLK, 128), lambda i: (i, 0))]
        + [bs_w(a) for a in ws],
        out_specs=[pl.BlockSpec((RBLK, NF_EMB), lambda i: (i, 0)),
                   pl.BlockSpec((RBLK, 16), lambda i: (i, 0)),
                   pl.BlockSpec((1, NF_EMB), lambda i: (0, 0))],
        out_shape=[jax.ShapeDtypeStruct((N, NF_EMB), jnp.float32),
                   jax.ShapeDtypeStruct((N, 16), jnp.float32),
                   jax.ShapeDtypeStruct((1, NF_EMB), jnp.float32)],
    )(feats, acc0, acc1, coors, hfw, *ws)


def kernel(x, edge_attr, pos, params, edge_index, batch):
    loops = jnp.arange(N, dtype=edge_index.dtype)
    pad = E_PAD - E
    src = jnp.concatenate([edge_index[0], loops,
                           jnp.zeros((pad,), jnp.int32)]).reshape(NW, G, CH)
    dst = jnp.concatenate([edge_index[1], loops,
                           jnp.zeros((pad,), jnp.int32)]).reshape(NW, G, CH)
    ea = jnp.concatenate(
        [edge_attr, jnp.zeros((E_PAD - E0, EDGE_ATTR_DIM), jnp.float32)], axis=0)
    coors = jnp.concatenate([pos, jnp.zeros((N, 13), jnp.float32)], axis=1)
    zeros48 = jnp.zeros((N_PAD, 128), jnp.float32)

    embs = [params['emb_%d' % i] for i in range(len(EMB_IDXS))]
    feats, colsum = _init_call(x, embs)

    for p in params['layers']:
        wi = p['e_w1'][:NF_EMB]
        wj = p['e_w1'][NF_EMB:2 * NF_EMB]
        wa = p['e_w1'][2 * NF_EMB:2 * NF_EMB + EDGE_ATTR_DIM]
        wdrow = p['e_w1'][2 * NF_EMB + EDGE_ATTR_DIM:]
        cw2p = jnp.concatenate(
            [p['c_w2'], jnp.zeros((32, 15), jnp.float32)], axis=1)
        cb2p = jnp.concatenate(
            [p['c_b2'], jnp.zeros((15,), jnp.float32)]).reshape(1, 16)

        s2 = _var_call(feats, colsum)
        tdst, tsrc, hfw = _tables_call(
            feats, coors, colsum, s2, p['ln_w'].reshape(1, -1),
            p['ln_b'].reshape(1, -1), wi, wj, p['n_w1'][:NF_EMB],
            p['n_b1'].reshape(1, -1))
        gd, gs = _gather_call(tdst, tsrc, dst, src)
        m = _edge_call(gd, gs, ea, wa, p['e_b1'].reshape(1, -1), wdrow,
                       p['e_w2'], p['e_b2'].reshape(1, -1), p['c_w1'],
                       p['c_b1'].reshape(1, -1), cw2p, cb2p)
        accp = _scatter_call(m, dst, zeros48)
        feats, coors, colsum = _update_call(
            feats, accp[0], accp[1], coors, hfw, p['n_w1'][NF_EMB:],
            p['n_w2'], p['n_b2'].reshape(1, -1))

    return colsum
